# baseline (speedup 1.0000x reference)
import functools

import jax
import jax.numpy as jnp
from jax import lax
from jax.experimental import pallas as pl
from jax.experimental.pallas import tpu as pltpu
from jax.experimental.pallas import tpu_sc as plsc

N = 10000
E = 320000
D = 128

NC = 2
NS = 16
NW = NC * NS
LANES = 16
CH = 128
EPT = 10240
NCH = EPT // CH
SC4 = 4
NSC = NCH // SC4
ET = NW * EPT
NPAD = 10240
SPT = NPAD // NS


def _proj_body(h_ref, wfc_ref, w8_ref, z_ref, at_ref):
    z = lax.dot_general(h_ref[...], wfc_ref[...], (((1,), (1,)), ((), ())),
                        preferred_element_type=jnp.float32)
    z_ref[...] = z
    at_ref[...] = lax.dot_general(w8_ref[...], z, (((1,), (1,)), ((), ())),
                                  preferred_element_type=jnp.float32)


def _tc_proj(h, wfc, w8):
    return pl.pallas_call(
        _proj_body,
        out_shape=[
            jax.ShapeDtypeStruct((N, D), jnp.float32),
            jax.ShapeDtypeStruct((8, N), jnp.float32),
        ],
    )(h, wfc, w8)


def _sc_scores(src, dst, asrc, adst):
    mesh = plsc.VectorSubcoreMesh(core_axis_name="c", subcore_axis_name="s")

    @functools.partial(
        pl.kernel,
        mesh=mesh,
        out_type=[
            jax.ShapeDtypeStruct((ET,), jnp.float32),
            jax.ShapeDtypeStruct((NC * NPAD,), jnp.float32),
        ],
        scratch_types=[
            pltpu.VMEM((EPT,), jnp.int32),
            pltpu.VMEM((EPT,), jnp.int32),
            pltpu.VMEM((N,), jnp.float32),
            pltpu.VMEM((N,), jnp.float32),
            pltpu.VMEM((EPT,), jnp.float32),
            pltpu.VMEM((CH,), jnp.int32),
            pltpu.VMEM((NPAD,), jnp.float32),
            pltpu.VMEM_SHARED((NPAD,), jnp.float32),
        ],
        compiler_params=pltpu.CompilerParams(needs_layout_passes=False),
    )
    def k(src_hbm, dst_hbm, asrc_hbm, adst_hbm, s_hbm, den_hbm,
          src_v, dst_v, asrc_v, adst_v, s_v, idx_c, zero_v, den_sh):
        cid = lax.axis_index("c")
        sid = lax.axis_index("s")
        wid = sid * NC + cid
        base = wid * EPT
        pltpu.sync_copy(src_hbm.at[pl.ds(base, EPT)], src_v)
        pltpu.sync_copy(dst_hbm.at[pl.ds(base, EPT)], dst_v)
        pltpu.sync_copy(asrc_hbm, asrc_v)
        pltpu.sync_copy(adst_hbm, adst_v)

        zf = jnp.zeros((LANES,), jnp.float32)

        def zbody(i, c):
            zero_v[pl.ds(i * LANES, LANES)] = zf
            return c
        lax.fori_loop(0, NPAD // LANES, zbody, 0)

        lane = lax.iota(jnp.int32, LANES)

        def sbody(i, c):
            sl = pl.ds(i * LANES, LANES)
            av = plsc.load_gather(asrc_v, [src_v[sl]])
            bv = plsc.load_gather(adst_v, [dst_v[sl]])
            x = av + bv
            x = jnp.maximum(x, 0.01 * x)
            s = jnp.exp(x)
            g = base + i * LANES + lane
            s_v[sl] = jnp.where(g < E, s, 0.0)
            return c
        lax.fori_loop(0, EPT // LANES, sbody, 0)

        @pl.when(sid == 0)
        def _():
            pltpu.sync_copy(zero_v, den_sh)
        plsc.subcore_barrier()

        def scat(c, carry):
            cb = c * CH
            for t in range(CH // LANES):
                idx_c[pl.ds(t * LANES, LANES)] = dst_v[pl.ds(cb + t * LANES, LANES)]
            pltpu.sync_copy(s_v.at[pl.ds(cb, CH)], den_sh.at[idx_c], add=True)
            return carry
        lax.fori_loop(0, NCH, scat, 0)
        plsc.subcore_barrier()

        @pl.when(sid == 0)
        def _():
            pltpu.sync_copy(den_sh, den_hbm.at[pl.ds(cid * NPAD, NPAD)])
        pltpu.sync_copy(s_v, s_hbm.at[pl.ds(base, EPT)])

    return k(src, dst, asrc, adst)


def _sc_aggregate(z, src, dst, s):
    mesh = plsc.VectorSubcoreMesh(core_axis_name="c", subcore_axis_name="s")

    @functools.partial(
        pl.kernel,
        mesh=mesh,
        out_type=jax.ShapeDtypeStruct((NC, NPAD, D), jnp.float32),
        scratch_types=[
            pltpu.VMEM((SC4 * CH,), jnp.int32),
            pltpu.VMEM((SC4 * CH,), jnp.int32),
            pltpu.VMEM((SC4, CH), jnp.int32),
            pltpu.VMEM((SC4, CH), jnp.int32),
            pltpu.VMEM((SC4 * CH,), jnp.float32),
            pltpu.VMEM((SC4 * CH,), jnp.float32),
            pltpu.VMEM((CH, D), jnp.float32),
            pltpu.VMEM((CH, D), jnp.float32),
            pltpu.VMEM_SHARED((NPAD, D), jnp.float32),
            pltpu.SemaphoreType.DMA,
            pltpu.SemaphoreType.DMA,
            pltpu.SemaphoreType.DMA,
            pltpu.SemaphoreType.DMA,
            pltpu.SemaphoreType.DMA,
            pltpu.SemaphoreType.DMA,
        ],
        compiler_params=pltpu.CompilerParams(needs_layout_passes=False),
    )
    def k(z_hbm, src_hbm, dst_hbm, s_hbm, out_hbm,
          sidxA, sidxB, didxA, didxB, scA, scB, rows0, rows1, out_sh,
          isemA, isemB, gsem0, gsem1, ssem0, ssem1):
        cid = lax.axis_index("c")
        sid = lax.axis_index("s")
        wid = sid * NC + cid
        base = wid * EPT

        zf = jnp.zeros((LANES,), jnp.float32)
        zi = jnp.zeros((LANES,), jnp.int32)
        islot = ((sidxA, didxA, scA, isemA), (sidxB, didxB, scB, isemB))
        rslot = ((rows0, gsem0, ssem0), (rows1, gsem1, ssem1))

        def stage_idx(p, s_):
            sidx, didx, scv, isem = islot[s_]
            gb = base + p * SC4 * CH
            pltpu.async_copy(src_hbm.at[pl.ds(gb, SC4 * CH)], sidx, isem)
            pltpu.async_copy(s_hbm.at[pl.ds(gb, SC4 * CH)], scv, isem)
            for r in range(SC4):
                pltpu.async_copy(dst_hbm.at[pl.ds(gb + r * CH, CH)],
                                 didx.at[r], isem)

        def wait_idx(s_):
            sidx, didx, scv, isem = islot[s_]
            pltpu.make_async_copy(src_hbm.at[pl.ds(0, SC4 * CH)], sidx,
                                  isem).wait()
            pltpu.make_async_copy(s_hbm.at[pl.ds(0, SC4 * CH)], scv,
                                  isem).wait()
            for r in range(SC4):
                pltpu.make_async_copy(dst_hbm.at[pl.ds(0, CH)], didx.at[r],
                                      isem).wait()

        def gather(j, s_):
            sidx, _, _, _ = islot[s_]
            rows, gsem, ssem = rslot[j % 2]
            pltpu.make_async_copy(z_hbm.at[pl.ds(0, CH)], rows, ssem).wait()
            pltpu.async_copy(z_hbm.at[sidx.at[pl.ds((j % SC4) * CH, CH)]],
                             rows, gsem)

        def proc(j, s_):
            _, didx, scv, _ = islot[s_]
            rows, gsem, ssem = rslot[j % 2]
            jj = j % SC4
            pltpu.make_async_copy(z_hbm.at[pl.ds(0, CH)], rows, gsem).wait()

            def ebody(q, c2):
                for u in range(4):
                    e = q * 4 + u
                    ab = plsc.load_gather(
                        scv, [jnp.full((LANES,), jj * CH + e, jnp.int32)])
                    for t in range(D // LANES):
                        sl = pl.ds(t * LANES, LANES)
                        rows[e, sl] = rows[e, sl] * ab
                return c2
            lax.fori_loop(0, CH // 4, ebody, 0)
            pltpu.async_copy(rows, out_sh.at[didx.at[jj]], ssem, add=True)

        def zrow(r, c):
            for t in range(D // LANES):
                rows0[r, pl.ds(t * LANES, LANES)] = zf
                rows1[r, pl.ds(t * LANES, LANES)] = zf
            return c
        lax.fori_loop(0, CH, zrow, 0)
        for t in range(CH // LANES):
            didxA[0, pl.ds(t * LANES, LANES)] = zi
            didxB[0, pl.ds(t * LANES, LANES)] = zi
        for q in range(SPT // CH):
            pltpu.sync_copy(rows0, out_sh.at[pl.ds(sid * SPT + q * CH, CH)])
        plsc.subcore_barrier()

        pltpu.async_copy(rows0, out_sh.at[didxA.at[0]], ssem0, add=True)
        pltpu.async_copy(rows1, out_sh.at[didxB.at[0]], ssem1, add=True)

        stage_idx(0, 0)
        wait_idx(0)
        gather(0, 0)

        def body(pp, carry):
            gather(1, 0)
            stage_idx(2 * pp + 1, 1)
            proc(0, 0)
            gather(2, 0)
            proc(1, 0)
            gather(3, 0)
            proc(2, 0)
            wait_idx(1)
            gather(4, 1)
            proc(3, 0)
            gather(5, 1)

            @pl.when(pp + 1 < NSC // 2)
            def _():
                stage_idx(2 * pp + 2, 0)
            proc(4, 1)
            gather(6, 1)
            proc(5, 1)
            gather(7, 1)
            proc(6, 1)

            @pl.when(pp + 1 < NSC // 2)
            def _():
                wait_idx(0)
                gather(8, 0)
            proc(7, 1)
            return carry
        lax.fori_loop(0, NSC // 2, body, 0)
        pltpu.make_async_copy(z_hbm.at[pl.ds(0, CH)], rows0, ssem0).wait()
        pltpu.make_async_copy(z_hbm.at[pl.ds(0, CH)], rows1, ssem1).wait()
        plsc.subcore_barrier()

        pltpu.sync_copy(out_sh.at[pl.ds(sid * SPT, SPT)],
                        out_hbm.at[cid, pl.ds(sid * SPT, SPT)])

    return k(z, src, dst, s)


def _finish_body(a_ref, b_ref, dena_ref, denb_ref, o_ref):
    br = o_ref.shape[0]
    d = dena_ref[0, 0, 0] + denb_ref[0, 0, 0]
    d = jnp.where(d == 0.0, 1.0, d)
    inv = jnp.reshape(1.0 / d, (br, 1))
    o_ref[...] = (a_ref[0] + b_ref[0]) * inv


def _tc_finish(parts, denp):
    br = 1024
    den4 = jnp.reshape(denp, (NC, NPAD // br, 1, br))
    return pl.pallas_call(
        _finish_body,
        grid=(NPAD // br,),
        in_specs=[
            pl.BlockSpec((1, br, D), lambda i: (0, i, 0)),
            pl.BlockSpec((1, br, D), lambda i: (1, i, 0)),
            pl.BlockSpec((1, 1, 1, br), lambda i: (0, i, 0, 0)),
            pl.BlockSpec((1, 1, 1, br), lambda i: (1, i, 0, 0)),
        ],
        out_specs=pl.BlockSpec((br, D), lambda i: (i, 0)),
        out_shape=jax.ShapeDtypeStruct((NPAD, D), jnp.float32),
    )(parts, parts, den4, den4)


def kernel(h, edge_index, W_fc, W_attn):
    src = jnp.pad(edge_index[0].astype(jnp.int32), (0, ET - E))
    dst = jnp.pad(edge_index[1].astype(jnp.int32), (0, ET - E))
    w8 = jnp.zeros((8, D), jnp.float32)
    w8 = w8.at[0].set(W_attn[0, :D]).at[1].set(W_attn[0, D:])
    z, at = _tc_proj(h, W_fc, w8)
    s, denp = _sc_scores(src, dst, at[0], at[1])
    parts = _sc_aggregate(z, src, dst, s)
    return _tc_finish(parts, denp)[:N]
